# Initial kernel scaffold; baseline (speedup 1.0000x reference)
import jax
import jax.numpy as jnp
from jax.experimental import pallas as pl


def kernel(indexes, embedding_weight, head_weight):
    raise NotImplementedError("write your pallas kernel here")


# speedup vs baseline: 1.1660x; 1.1660x over previous
import functools

import jax
import jax.numpy as jnp
from jax import lax
from jax.experimental import pallas as pl
from jax.experimental.pallas import tpu as pltpu
from jax.experimental.pallas import tpu_sc as plsc

_NUM_CORES = 2
_NUM_SUBCORES = 16
_NW = _NUM_CORES * _NUM_SUBCORES
_LANES = 16

_CHUNK_BAGS = 4


def _sc_bag_sum(idx3, table, batch, hist, dim):
    b_per_w = batch // _NW
    n_chunks = b_per_w // _CHUNK_BAGS
    rows_per_chunk = _CHUNK_BAGS * hist

    mesh = plsc.VectorSubcoreMesh(
        core_axis_name="c", subcore_axis_name="s",
        num_cores=_NUM_CORES, num_subcores=_NUM_SUBCORES)

    @functools.partial(
        pl.kernel,
        out_type=jax.ShapeDtypeStruct((batch, dim), jnp.float32),
        mesh=mesh,
        scratch_types=[
            pltpu.VMEM((n_chunks, rows_per_chunk), jnp.int32),
            pltpu.VMEM((rows_per_chunk, dim), jnp.float32),
            pltpu.VMEM((b_per_w, dim), jnp.float32),
            pltpu.SemaphoreType.DMA,
        ],
    )
    def bag_sum(idx_hbm, table_hbm, out_hbm, idx_v, rows_v, out_v, sem):
        wid = lax.axis_index("s") * _NUM_CORES + lax.axis_index("c")
        pltpu.sync_copy(idx_hbm.at[wid], idx_v)

        def chunk_body(c, carry):
            pltpu.async_copy(table_hbm.at[idx_v.at[c]], rows_v, sem).wait()
            for bag in range(_CHUNK_BAGS):
                for v in range(dim // _LANES):
                    sl = pl.ds(v * _LANES, _LANES)
                    acc = rows_v[bag * hist, sl]
                    for r in range(1, hist):
                        acc = acc + rows_v[bag * hist + r, sl]
                    out_v[c * _CHUNK_BAGS + bag, sl] = acc
            return carry

        lax.fori_loop(0, n_chunks, chunk_body, 0, unroll=False)
        pltpu.sync_copy(out_v, out_hbm.at[pl.ds(wid * b_per_w, b_per_w)])

    return bag_sum(idx3, table)


def _tc_head(summed, idx, wt_bf16, batch, hist, dim, classes, tb):
    def body(summed_ref, idx_ref, wt_ref, out_ref):
        cnt = jnp.sum((idx_ref[...] != 0).astype(jnp.float32), axis=1,
                      keepdims=True)
        pooled = summed_ref[...] / jnp.maximum(cnt, 1.0)
        logits = jnp.dot(pooled.astype(jnp.bfloat16), wt_ref[...],
                         preferred_element_type=jnp.float32)
        m = jnp.max(logits, axis=1, keepdims=True)
        shifted = logits - m
        lse = jnp.log(jnp.sum(jnp.exp(shifted), axis=1, keepdims=True))
        out_ref[...] = shifted - lse

    grid = (batch // tb,)
    return pl.pallas_call(
        body,
        grid=grid,
        in_specs=[
            pl.BlockSpec((tb, dim), lambda i: (i, 0)),
            pl.BlockSpec((tb, hist), lambda i: (i, 0)),
            pl.BlockSpec((dim, classes), lambda i: (0, 0)),
        ],
        out_specs=pl.BlockSpec((tb, classes), lambda i: (i, 0)),
        out_shape=jax.ShapeDtypeStruct((batch, classes), jnp.float32),
        compiler_params=pltpu.CompilerParams(
            dimension_semantics=("arbitrary",),
            vmem_limit_bytes=100 * 1024 * 1024,
        ),
    )(summed, idx, wt_bf16)


def kernel(indexes, embedding_weight, head_weight):
    batch, hist = indexes.shape
    vocab, dim = embedding_weight.shape
    classes = head_weight.shape[0]

    idx = indexes.astype(jnp.int32)
    b_per_w = batch // _NW
    n_chunks = b_per_w // _CHUNK_BAGS
    idx3 = idx.reshape(_NW, n_chunks, _CHUNK_BAGS * hist)

    summed = _sc_bag_sum(idx3, embedding_weight, batch, hist, dim)

    wt = head_weight.T.astype(jnp.bfloat16)
    return _tc_head(summed, idx, wt, batch, hist, dim, classes, tb=256)


# speedup vs baseline: 1.1671x; 1.0010x over previous
import functools

import jax
import jax.numpy as jnp
from jax import lax
from jax.experimental import pallas as pl
from jax.experimental.pallas import tpu as pltpu
from jax.experimental.pallas import tpu_sc as plsc

_NUM_CORES = 2
_NUM_SUBCORES = 16
_NW = _NUM_CORES * _NUM_SUBCORES
_LANES = 16

_CHUNK_BAGS = 4


def _sc_bag_sum(idx3, table, batch, hist, dim):
    b_per_w = batch // _NW
    n_chunks = b_per_w // _CHUNK_BAGS
    rows_per_chunk = _CHUNK_BAGS * hist

    mesh = plsc.VectorSubcoreMesh(
        core_axis_name="c", subcore_axis_name="s",
        num_cores=_NUM_CORES, num_subcores=_NUM_SUBCORES)

    @functools.partial(
        pl.kernel,
        out_type=jax.ShapeDtypeStruct((batch, dim), jnp.float32),
        mesh=mesh,
        scratch_types=[
            pltpu.VMEM((n_chunks, rows_per_chunk), jnp.int32),
            pltpu.VMEM((rows_per_chunk, dim), jnp.float32),
            pltpu.VMEM((b_per_w, dim), jnp.float32),
            pltpu.SemaphoreType.DMA,
        ],
    )
    def bag_sum(idx_hbm, table_hbm, out_hbm, idx_v, rows_v, out_v, sem):
        wid = lax.axis_index("s") * _NUM_CORES + lax.axis_index("c")
        pltpu.sync_copy(idx_hbm.at[wid], idx_v)

        def chunk_body(c, carry):
            pltpu.async_copy(table_hbm.at[idx_v.at[c]], rows_v, sem).wait()
            for bag in range(_CHUNK_BAGS):
                for v in range(dim // _LANES):
                    sl = pl.ds(v * _LANES, _LANES)
                    acc = rows_v[bag * hist, sl]
                    for r in range(1, hist):
                        acc = acc + rows_v[bag * hist + r, sl]
                    out_v[c * _CHUNK_BAGS + bag, sl] = acc
            return carry

        lax.fori_loop(0, n_chunks, chunk_body, 0, unroll=False)
        pltpu.sync_copy(out_v, out_hbm.at[pl.ds(wid * b_per_w, b_per_w)])

    return bag_sum(idx3, table)


def _tc_head(summed, idx_pad, wt_bf16, batch, dim, classes, tb):
    def body(summed_ref, idx_ref, wt_ref, out_ref):
        cnt = jnp.sum((idx_ref[...] != 0).astype(jnp.float32), axis=1,
                      keepdims=True)
        pooled = summed_ref[...] / jnp.maximum(cnt, 1.0)
        logits = lax.dot_general(
            pooled.astype(jnp.bfloat16), wt_ref[...],
            (((1,), (1,)), ((), ())),
            preferred_element_type=jnp.float32)
        m = jnp.max(logits, axis=1, keepdims=True)
        shifted = logits - m
        lse = jnp.log(jnp.sum(jnp.exp(shifted), axis=1, keepdims=True))
        out_ref[...] = shifted - lse

    grid = (batch // tb,)
    return pl.pallas_call(
        body,
        grid=grid,
        in_specs=[
            pl.BlockSpec((tb, dim), lambda i: (i, 0)),
            pl.BlockSpec((tb, 128), lambda i: (i, 0)),
            pl.BlockSpec((classes, dim), lambda i: (0, 0)),
        ],
        out_specs=pl.BlockSpec((tb, classes), lambda i: (i, 0)),
        out_shape=jax.ShapeDtypeStruct((batch, classes), jnp.float32),
        compiler_params=pltpu.CompilerParams(
            dimension_semantics=("arbitrary",),
            vmem_limit_bytes=100 * 1024 * 1024,
        ),
    )(summed, idx_pad, wt_bf16)


def kernel(indexes, embedding_weight, head_weight):
    batch, hist = indexes.shape
    vocab, dim = embedding_weight.shape
    classes = head_weight.shape[0]

    idx = indexes.astype(jnp.int32)
    b_per_w = batch // _NW
    n_chunks = b_per_w // _CHUNK_BAGS
    idx3 = idx.reshape(_NW, n_chunks, _CHUNK_BAGS * hist)

    summed = _sc_bag_sum(idx3, embedding_weight, batch, hist, dim)

    idx_pad = jnp.pad(idx, ((0, 0), (0, 128 - hist)))
    wt = head_weight.astype(jnp.bfloat16)
    return _tc_head(summed, idx_pad, wt, batch, dim, classes, tb=256)


# speedup vs baseline: 1.9110x; 1.6374x over previous
import functools

import jax
import jax.numpy as jnp
from jax import lax
from jax.experimental import pallas as pl
from jax.experimental.pallas import tpu as pltpu
from jax.experimental.pallas import tpu_sc as plsc

_NUM_CORES = 2
_NUM_SUBCORES = 16
_NW = _NUM_CORES * _NUM_SUBCORES
_LANES = 16

_CHUNK_BAGS = 4


def _sc_bag_sum(idx3, table, batch, hist, dim):
    b_per_w = batch // _NW
    n_chunks = b_per_w // _CHUNK_BAGS
    rows_per_chunk = _CHUNK_BAGS * hist

    mesh = plsc.VectorSubcoreMesh(
        core_axis_name="c", subcore_axis_name="s",
        num_cores=_NUM_CORES, num_subcores=_NUM_SUBCORES)

    @functools.partial(
        pl.kernel,
        out_type=jax.ShapeDtypeStruct((batch, dim), jnp.float32),
        mesh=mesh,
        scratch_types=[
            pltpu.VMEM((n_chunks, rows_per_chunk), jnp.int32),
            pltpu.VMEM((rows_per_chunk, dim), jnp.float32),
            pltpu.VMEM((b_per_w, dim), jnp.float32),
            pltpu.SemaphoreType.DMA,
        ],
    )
    def bag_sum(idx_hbm, table_hbm, out_hbm, idx_v, rows_v, out_v, sem):
        wid = lax.axis_index("s") * _NUM_CORES + lax.axis_index("c")
        pltpu.sync_copy(idx_hbm.at[wid], idx_v)

        def chunk_body(c, carry):
            pltpu.async_copy(table_hbm.at[idx_v.at[c]], rows_v, sem).wait()
            for bag in range(_CHUNK_BAGS):
                for v in range(dim // _LANES):
                    sl = pl.ds(v * _LANES, _LANES)
                    acc = rows_v[bag * hist, sl]
                    for r in range(1, hist):
                        acc = acc + rows_v[bag * hist + r, sl]
                    out_v[c * _CHUNK_BAGS + bag, sl] = acc
            return carry

        lax.fori_loop(0, n_chunks, chunk_body, 0, unroll=False)
        pltpu.sync_copy(out_v, out_hbm.at[pl.ds(wid * b_per_w, b_per_w)])

    return bag_sum(idx3, table)


def _tc_head(summed, idxT, wt_bf16, batch, hist, dim, classes, tb):
    def body(summed_ref, idxT_ref, wt_ref, out_ref):
        cnt = jnp.sum((idxT_ref[...] != 0).astype(jnp.float32), axis=0,
                      keepdims=True)
        recip = 1.0 / jnp.maximum(cnt, 1.0)
        logits = lax.dot_general(
            wt_ref[...], summed_ref[...].astype(jnp.bfloat16),
            (((1,), (1,)), ((), ())),
            preferred_element_type=jnp.float32) * recip
        m = jnp.max(logits, axis=0, keepdims=True)
        shifted = logits - m
        lse = jnp.log(jnp.sum(jnp.exp(shifted), axis=0, keepdims=True))
        out_ref[...] = shifted - lse

    grid = (batch // tb,)
    return pl.pallas_call(
        body,
        grid=grid,
        in_specs=[
            pl.BlockSpec((tb, dim), lambda i: (i, 0)),
            pl.BlockSpec((hist, tb), lambda i: (0, i)),
            pl.BlockSpec((classes, dim), lambda i: (0, 0)),
        ],
        out_specs=pl.BlockSpec((classes, tb), lambda i: (0, i)),
        out_shape=jax.ShapeDtypeStruct((classes, batch), jnp.float32),
        compiler_params=pltpu.CompilerParams(
            dimension_semantics=("arbitrary",),
            vmem_limit_bytes=100 * 1024 * 1024,
        ),
    )(summed, idxT, wt_bf16)


def kernel(indexes, embedding_weight, head_weight):
    batch, hist = indexes.shape
    vocab, dim = embedding_weight.shape
    classes = head_weight.shape[0]

    idx = indexes.astype(jnp.int32)
    b_per_w = batch // _NW
    n_chunks = b_per_w // _CHUNK_BAGS
    idx3 = idx.reshape(_NW, n_chunks, _CHUNK_BAGS * hist)

    summed = _sc_bag_sum(idx3, embedding_weight, batch, hist, dim)

    wt = head_weight.astype(jnp.bfloat16)
    out_t = _tc_head(summed, idx.T, wt, batch, hist, dim, classes, tb=256)
    return out_t.T


# speedup vs baseline: 2.0600x; 1.0780x over previous
import functools

import jax
import jax.numpy as jnp
from jax import lax
from jax.experimental import pallas as pl
from jax.experimental.pallas import tpu as pltpu
from jax.experimental.pallas import tpu_sc as plsc

_NUM_CORES = 2
_NUM_SUBCORES = 16
_NW = _NUM_CORES * _NUM_SUBCORES
_LANES = 16

_CHUNK_BAGS = 4


def _sc_bag_sum(idx3, table, batch, hist, dim):
    b_per_w = batch // _NW
    n_chunks = b_per_w // _CHUNK_BAGS
    rows_per_chunk = _CHUNK_BAGS * hist

    mesh = plsc.VectorSubcoreMesh(
        core_axis_name="c", subcore_axis_name="s",
        num_cores=_NUM_CORES, num_subcores=_NUM_SUBCORES)

    @functools.partial(
        pl.kernel,
        out_type=jax.ShapeDtypeStruct((batch, dim), jnp.float32),
        mesh=mesh,
        scratch_types=[
            pltpu.VMEM((n_chunks, rows_per_chunk), jnp.int32),
            pltpu.VMEM((rows_per_chunk, dim), jnp.float32),
            pltpu.VMEM((rows_per_chunk, dim), jnp.float32),
            pltpu.VMEM((b_per_w, dim), jnp.float32),
            pltpu.SemaphoreType.DMA,
            pltpu.SemaphoreType.DMA,
        ],
    )
    def bag_sum(idx_hbm, table_hbm, out_hbm, idx_v, rows0, rows1, out_v,
                sem0, sem1):
        wid = lax.axis_index("s") * _NUM_CORES + lax.axis_index("c")
        pltpu.sync_copy(idx_hbm.at[wid], idx_v)

        def accum(c, rows_v):
            for bag in range(_CHUNK_BAGS):
                for v in range(dim // _LANES):
                    sl = pl.ds(v * _LANES, _LANES)
                    vals = [rows_v[bag * hist + r, sl] for r in range(hist)]
                    while len(vals) > 1:
                        nxt = [a + b for a, b in zip(vals[::2], vals[1::2])]
                        if len(vals) % 2:
                            nxt.append(vals[-1])
                        vals = nxt
                    out_v[c * _CHUNK_BAGS + bag, sl] = vals[0]

        pltpu.async_copy(table_hbm.at[idx_v.at[0]], rows0, sem0)

        def chunk_body(p, carry):
            c = p * 2
            pltpu.async_copy(table_hbm.at[idx_v.at[c + 1]], rows1, sem1)
            pltpu.make_async_copy(table_hbm.at[idx_v.at[c]], rows0,
                                  sem0).wait()
            accum(c, rows0)

            @pl.when(p + 1 < n_chunks // 2)
            def _():
                pltpu.async_copy(table_hbm.at[idx_v.at[c + 2]], rows0, sem0)

            pltpu.make_async_copy(table_hbm.at[idx_v.at[c + 1]], rows1,
                                  sem1).wait()
            accum(c + 1, rows1)
            return carry

        lax.fori_loop(0, n_chunks // 2, chunk_body, 0, unroll=False)
        pltpu.sync_copy(out_v, out_hbm.at[pl.ds(wid * b_per_w, b_per_w)])

    return bag_sum(idx3, table)


def _tc_head(summed, idxT, wt_bf16, batch, hist, dim, classes, tb):
    def body(summed_ref, idxT_ref, wt_ref, out_ref):
        cnt = jnp.sum((idxT_ref[...] != 0).astype(jnp.float32), axis=0,
                      keepdims=True)
        recip = 1.0 / jnp.maximum(cnt, 1.0)
        logits = lax.dot_general(
            wt_ref[...], summed_ref[...].astype(jnp.bfloat16),
            (((1,), (1,)), ((), ())),
            preferred_element_type=jnp.float32) * recip
        m = jnp.max(logits, axis=0, keepdims=True)
        shifted = logits - m
        lse = jnp.log(jnp.sum(jnp.exp(shifted), axis=0, keepdims=True))
        out_ref[...] = shifted - lse

    grid = (batch // tb,)
    return pl.pallas_call(
        body,
        grid=grid,
        in_specs=[
            pl.BlockSpec((tb, dim), lambda i: (i, 0)),
            pl.BlockSpec((hist, tb), lambda i: (0, i)),
            pl.BlockSpec((classes, dim), lambda i: (0, 0)),
        ],
        out_specs=pl.BlockSpec((classes, tb), lambda i: (0, i)),
        out_shape=jax.ShapeDtypeStruct((classes, batch), jnp.float32),
        compiler_params=pltpu.CompilerParams(
            dimension_semantics=("arbitrary",),
            vmem_limit_bytes=100 * 1024 * 1024,
        ),
    )(summed, idxT, wt_bf16)


def kernel(indexes, embedding_weight, head_weight):
    batch, hist = indexes.shape
    vocab, dim = embedding_weight.shape
    classes = head_weight.shape[0]

    idx = indexes.astype(jnp.int32)
    b_per_w = batch // _NW
    n_chunks = b_per_w // _CHUNK_BAGS
    idx3 = idx.reshape(_NW, n_chunks, _CHUNK_BAGS * hist)

    summed = _sc_bag_sum(idx3, embedding_weight, batch, hist, dim)

    wt = head_weight.astype(jnp.bfloat16)
    out_t = _tc_head(summed, idx.T, wt, batch, hist, dim, classes, tb=256)
    return out_t.T


# speedup vs baseline: 2.1419x; 1.0398x over previous
import functools

import jax
import jax.numpy as jnp
from jax import lax
from jax.experimental import pallas as pl
from jax.experimental.pallas import tpu as pltpu
from jax.experimental.pallas import tpu_sc as plsc

_NUM_CORES = 2
_NUM_SUBCORES = 16
_NW = _NUM_CORES * _NUM_SUBCORES
_LANES = 16

_CHUNK_BAGS = 4


def _sc_bag_sum(idx3, table, batch, hist, dim):
    b_per_w = batch // _NW
    n_chunks = b_per_w // _CHUNK_BAGS
    rows_per_chunk = _CHUNK_BAGS * hist

    mesh = plsc.VectorSubcoreMesh(
        core_axis_name="c", subcore_axis_name="s",
        num_cores=_NUM_CORES, num_subcores=_NUM_SUBCORES)

    @functools.partial(
        pl.kernel,
        out_type=jax.ShapeDtypeStruct((batch, dim), jnp.float32),
        mesh=mesh,
        scratch_types=[
            pltpu.VMEM((n_chunks, rows_per_chunk), jnp.int32),
            pltpu.VMEM((rows_per_chunk, dim), jnp.float32),
            pltpu.VMEM((rows_per_chunk, dim), jnp.float32),
            pltpu.VMEM((b_per_w, dim), jnp.float32),
            pltpu.SemaphoreType.DMA,
            pltpu.SemaphoreType.DMA,
        ],
    )
    def bag_sum(idx_hbm, table_hbm, out_hbm, idx_v, rows0, rows1, out_v,
                sem0, sem1):
        wid = lax.axis_index("s") * _NUM_CORES + lax.axis_index("c")
        pltpu.sync_copy(idx_hbm.at[wid], idx_v)

        def accum(c, rows_v):
            for bag in range(_CHUNK_BAGS):
                for v in range(dim // _LANES):
                    sl = pl.ds(v * _LANES, _LANES)
                    vals = [rows_v[bag * hist + r, sl] for r in range(hist)]
                    while len(vals) > 1:
                        nxt = [a + b for a, b in zip(vals[::2], vals[1::2])]
                        if len(vals) % 2:
                            nxt.append(vals[-1])
                        vals = nxt
                    out_v[c * _CHUNK_BAGS + bag, sl] = vals[0]

        pltpu.async_copy(table_hbm.at[idx_v.at[0]], rows0, sem0)

        def chunk_body(p, carry):
            c = p * 2
            pltpu.async_copy(table_hbm.at[idx_v.at[c + 1]], rows1, sem1)
            pltpu.make_async_copy(table_hbm.at[idx_v.at[c]], rows0,
                                  sem0).wait()
            accum(c, rows0)

            @pl.when(p + 1 < n_chunks // 2)
            def _():
                pltpu.async_copy(table_hbm.at[idx_v.at[c + 2]], rows0, sem0)

            pltpu.make_async_copy(table_hbm.at[idx_v.at[c + 1]], rows1,
                                  sem1).wait()
            accum(c + 1, rows1)
            return carry

        lax.fori_loop(0, n_chunks // 2, chunk_body, 0, unroll=False)
        pltpu.sync_copy(out_v, out_hbm.at[pl.ds(wid * b_per_w, b_per_w)])

    return bag_sum(idx3, table)


def _tc_head(summed, idxT, wt_bf16, batch, hist, dim, classes, tb):
    def body(summed_ref, idxT_ref, wt_ref, out_ref):
        cnt = jnp.sum((idxT_ref[...] != 0).astype(jnp.float32), axis=0,
                      keepdims=True)
        recip = 1.0 / jnp.maximum(cnt, 1.0)
        pooledT = jnp.transpose(summed_ref[...]) * recip
        logits = lax.dot_general(
            wt_ref[...], pooledT.astype(jnp.bfloat16),
            (((1,), (0,)), ((), ())),
            preferred_element_type=jnp.float32)
        m = jnp.max(logits, axis=0, keepdims=True)
        shifted = logits - m
        lse = jnp.log(jnp.sum(jnp.exp(shifted), axis=0, keepdims=True))
        out_ref[...] = shifted - lse

    grid = (batch // tb,)
    return pl.pallas_call(
        body,
        grid=grid,
        in_specs=[
            pl.BlockSpec((tb, dim), lambda i: (i, 0)),
            pl.BlockSpec((hist, tb), lambda i: (0, i)),
            pl.BlockSpec((classes, dim), lambda i: (0, 0)),
        ],
        out_specs=pl.BlockSpec((classes, tb), lambda i: (0, i)),
        out_shape=jax.ShapeDtypeStruct((classes, batch), jnp.float32),
        compiler_params=pltpu.CompilerParams(
            dimension_semantics=("arbitrary",),
            vmem_limit_bytes=100 * 1024 * 1024,
        ),
    )(summed, idxT, wt_bf16)


def kernel(indexes, embedding_weight, head_weight):
    batch, hist = indexes.shape
    vocab, dim = embedding_weight.shape
    classes = head_weight.shape[0]

    idx = indexes.astype(jnp.int32)
    b_per_w = batch // _NW
    n_chunks = b_per_w // _CHUNK_BAGS
    idx3 = idx.reshape(_NW, n_chunks, _CHUNK_BAGS * hist)

    summed = _sc_bag_sum(idx3, embedding_weight, batch, hist, dim)

    wt = head_weight.astype(jnp.bfloat16)
    out_t = _tc_head(summed, idx.T, wt, batch, hist, dim, classes, tb=256)
    return out_t.T
